# baseline (speedup 1.0000x reference)
import functools

import jax
import jax.numpy as jnp
from jax import lax
from jax.experimental import pallas as pl
from jax.experimental.pallas import tpu as pltpu
from jax.experimental.pallas import tpu_sc as plsc

EPS = 1e-5

NC = 2
NS = 16

CB = 80
CSTG = 2000
ZROWS = 632


def _ln_body(x_ref, g_ref, b_ref, o_ref):
    x = x_ref[...]
    mu = jnp.mean(x, axis=-1, keepdims=True)
    xc = x - mu
    var = jnp.mean(xc * xc, axis=-1, keepdims=True)
    o_ref[...] = xc * lax.rsqrt(var + EPS) * g_ref[...] + b_ref[...]


def _layer_norm_tc(x, gamma, beta):
    n, d = x.shape
    blk = 1000
    return pl.pallas_call(
        _ln_body,
        grid=(n // blk,),
        in_specs=[
            pl.BlockSpec((blk, d), lambda i: (i, 0)),
            pl.BlockSpec((1, d), lambda i: (0, 0)),
            pl.BlockSpec((1, d), lambda i: (0, 0)),
        ],
        out_specs=pl.BlockSpec((blk, d), lambda i: (i, 0)),
        out_shape=jax.ShapeDtypeStruct((n, d), x.dtype),
    )(x, gamma.reshape(1, d), beta.reshape(1, d))


def _combine_body(acc_ref, cnt_ref, rel_ref, wc_ref, x0_ref, o_ref):
    cnt = cnt_ref[...]
    deg = jnp.sum(cnt, axis=1, keepdims=True)
    acc = acc_ref[0] + acc_ref[1]
    agg = acc + jnp.dot(cnt, rel_ref[...], preferred_element_type=jnp.float32)
    agg = agg / jnp.maximum(deg, 1.0)
    o_ref[...] = (
        jnp.dot(agg, wc_ref[...], preferred_element_type=jnp.float32)
        + x0_ref[...]
    )


def _combine_tc(acc2, cnt, rel, wc, x0):
    n, d = x0.shape
    r = rel.shape[0]
    blk = 1000
    return pl.pallas_call(
        _combine_body,
        grid=(n // blk,),
        in_specs=[
            pl.BlockSpec((NC, blk, d), lambda i: (0, i, 0)),
            pl.BlockSpec((blk, r), lambda i: (i, 0)),
            pl.BlockSpec((r, d), lambda i: (0, 0)),
            pl.BlockSpec((d, d), lambda i: (0, 0)),
            pl.BlockSpec((blk, d), lambda i: (i, 0)),
        ],
        out_specs=pl.BlockSpec((blk, d), lambda i: (i, 0)),
        out_shape=jax.ShapeDtypeStruct((n, d), jnp.float32),
    )(acc2, cnt, rel, wc, x0)


def _make_edge_sc(n, d, e, r):
    nh = n // NC
    ch = nh * r
    cnt_zpt = (ch + NS + NS * 8 - 1) // (NS * 8) * 8
    cnt_words = NS * cnt_zpt
    cnt_opt = ch // NS
    acc_a = (n // NS) // 8 * 8
    acc_last = n - acc_a * (NS - 1)
    ept = e // (NC * NS)
    nch = ept // CB
    npair = nch // 2

    mesh = plsc.VectorSubcoreMesh(core_axis_name="c", subcore_axis_name="s",
                                  num_cores=NC)

    @functools.partial(
        pl.kernel,
        out_type=(
            jax.ShapeDtypeStruct((NC, n, d), jnp.float32),
            jax.ShapeDtypeStruct((NC * ch,), jnp.float32),
        ),
        mesh=mesh,
        scratch_types=[
            pltpu.VMEM((CB,), jnp.int32),
            pltpu.VMEM((CB,), jnp.int32),
            pltpu.VMEM((CB,), jnp.int32),
            pltpu.VMEM((CB,), jnp.int32),
            pltpu.VMEM((CB,), jnp.int32),
            pltpu.VMEM((CB,), jnp.int32),
            pltpu.VMEM((CB,), jnp.int32),
            pltpu.VMEM((CB,), jnp.int32),
            pltpu.VMEM((CB, d), jnp.float32),
            pltpu.VMEM((CB, d), jnp.float32),
            pltpu.VMEM((CSTG,), jnp.float32),
            pltpu.VMEM((CSTG,), jnp.float32),
            pltpu.VMEM((CB,), jnp.float32),
            pltpu.VMEM_SHARED((n, d), jnp.float32),
            pltpu.VMEM_SHARED((cnt_words,), jnp.float32),
            pltpu.SemaphoreType.DMA,
            pltpu.SemaphoreType.DMA,
            pltpu.SemaphoreType.DMA,
            pltpu.SemaphoreType.DMA,
            pltpu.SemaphoreType.DMA,
        ],
    )
    def edge_kernel(h_hbm, src_hbm, fpk_hbm, zacc_hbm, zcnt_hbm,
                    ones_hbm, oacc_hbm, ocnt_hbm,
                    sidxA, sidxB, didxA, didxB, fA, fB, fmA, fmB,
                    rowsA, rowsB, cbuf, cbuf2, ones_v,
                    acc_sh, cnt_sh, semi, semgA, semgB, semsA, semsB):
        c = lax.axis_index("c")
        s = lax.axis_index("s")
        dump_cnt = ch + s
        lof = c * nh * r
        rsh = (r - 1).bit_length()
        tbase = (c * NS + s) * ept
        mbase = ((1 - c) * NS + s) * ept

        def stage(ch_i, si, f, fm):
            eb = tbase + ch_i * CB
            mb = mbase + ch_i * CB
            return [
                pltpu.async_copy(src_hbm.at[pl.ds(eb, CB)], si, semi),
                pltpu.async_copy(fpk_hbm.at[pl.ds(eb, CB)], f, semi),
                pltpu.async_copy(fpk_hbm.at[pl.ds(mb, CB)], fm, semi),
            ]

        def build(f, di):
            for k in range(CB // 16):
                sl = pl.ds(k * 16, 16)
                fv = f[sl]
                if di is not None:
                    di[sl] = lax.shift_right_logical(fv, rsh)
                t = fv - lof
                inb = (t >= 0) & (t < ch)
                f[sl] = jnp.where(inb, t, dump_cnt)

        def fire_gather(si, rows, semg):
            return pltpu.async_copy(h_hbm.at[si], rows, semg)

        def fire_scatter(rows, di, f2, fm, sems):
            return [
                pltpu.async_copy(rows, acc_sh.at[di], sems, add=True),
                pltpu.async_copy(ones_v, cnt_sh.at[f2], sems, add=True),
                pltpu.async_copy(ones_v, cnt_sh.at[fm], sems, add=True),
            ]

        stg0 = stage(0, sidxA, fA, fmA)

        @pl.when(s < NS - 1)
        def _():
            pltpu.async_copy(zacc_hbm, acc_sh.at[pl.ds(s * acc_a, ZROWS)],
                             semsA)

        @pl.when(s == NS - 1)
        def _():
            pltpu.async_copy(zacc_hbm, acc_sh.at[pl.ds((NS - 1) * acc_a,
                                                       ZROWS)], semsA)
            pltpu.async_copy(zacc_hbm.at[pl.ds(0, acc_last - ZROWS + 8)],
                             acc_sh.at[pl.ds((NS - 1) * acc_a + ZROWS - 8,
                                             acc_last - ZROWS + 8)], semsA)

        pltpu.sync_copy(zcnt_hbm, cbuf)
        nzc = cnt_zpt // CSTG
        zrem = cnt_zpt - nzc * CSTG
        zdesc = [
            pltpu.async_copy(
                cbuf, cnt_sh.at[pl.ds(s * cnt_zpt + p * CSTG, CSTG)], semsB)
            for p in range(nzc)
        ]
        if zrem:
            zdesc.append(pltpu.async_copy(
                cbuf.at[pl.ds(0, zrem)],
                cnt_sh.at[pl.ds(s * cnt_zpt + nzc * CSTG, zrem)], semsB))
        pltpu.sync_copy(ones_hbm, ones_v)
        for dsc in stg0:
            dsc.wait()
        fire_gather(sidxA, rowsA, semgA)
        pltpu.make_async_copy(zacc_hbm, acc_sh.at[pl.ds(s * acc_a, ZROWS)],
                              semsA).wait()

        @pl.when(s == NS - 1)
        def _():
            pltpu.make_async_copy(
                zacc_hbm.at[pl.ds(0, acc_last - ZROWS + 8)],
                acc_sh.at[pl.ds((NS - 1) * acc_a + ZROWS - 8,
                                acc_last - ZROWS + 8)], semsA).wait()

        for dsc in zdesc:
            dsc.wait()
        plsc.subcore_barrier()

        def body(i, _):
            even = 2 * i
            stg_o = stage(even + 1, sidxB, fB, fmB)
            build(fA, didxA)
            build(fmA, None)
            pltpu.make_async_copy(h_hbm.at[sidxA], rowsA, semgA).wait()
            sc_e = fire_scatter(rowsA, didxA, fA, fmA, semsA)
            for dsc in stg_o:
                dsc.wait()
            build(fB, didxB)
            build(fmB, None)
            fire_gather(sidxB, rowsB, semgB)
            for dsc in sc_e:
                dsc.wait()

            @pl.when(i < npair - 1)
            def _():
                for dsc in stage(even + 2, sidxA, fA, fmA):
                    dsc.wait()

            pltpu.make_async_copy(h_hbm.at[sidxB], rowsB, semgB).wait()
            sc_o = fire_scatter(rowsB, didxB, fB, fmB, semsB)

            @pl.when(i < npair - 1)
            def _():
                fire_gather(sidxA, rowsA, semgA)

            for dsc in sc_o:
                dsc.wait()
            return 0

        lax.fori_loop(0, npair, body, 0)

        if nch % 2:
            for dsc in stage(nch - 1, sidxA, fA, fmA):
                dsc.wait()
            build(fA, didxA)
            build(fmA, None)
            fire_gather(sidxA, rowsA, semgA).wait()
            for dsc in fire_scatter(rowsA, didxA, fA, fmA, semsA):
                dsc.wait()

        plsc.subcore_barrier()

        @pl.when(s < NS - 1)
        def _():
            pltpu.async_copy(acc_sh.at[pl.ds(s * acc_a, acc_a)],
                             oacc_hbm.at[c, pl.ds(s * acc_a, acc_a)], semgA)

        @pl.when(s == NS - 1)
        def _():
            pltpu.async_copy(acc_sh.at[pl.ds((NS - 1) * acc_a, acc_last)],
                             oacc_hbm.at[c, pl.ds((NS - 1) * acc_a,
                                                  acc_last)], semgA)

        npc = cnt_opt // CSTG
        wdesc = [None, None]
        for p in range(npc):
            buf = cbuf if p % 2 == 0 else cbuf2
            if wdesc[p % 2] is not None:
                wdesc[p % 2].wait()
            pltpu.sync_copy(
                cnt_sh.at[pl.ds(s * cnt_opt + p * CSTG, CSTG)], buf)
            wdesc[p % 2] = pltpu.async_copy(
                buf,
                ocnt_hbm.at[pl.ds(c * ch + s * cnt_opt + p * CSTG, CSTG)],
                semgB)
        for dsc in wdesc:
            if dsc is not None:
                dsc.wait()

        @pl.when(s < NS - 1)
        def _():
            pltpu.make_async_copy(
                acc_sh.at[pl.ds(s * acc_a, acc_a)],
                oacc_hbm.at[c, pl.ds(s * acc_a, acc_a)], semgA).wait()

        @pl.when(s == NS - 1)
        def _():
            pltpu.make_async_copy(
                acc_sh.at[pl.ds((NS - 1) * acc_a, acc_last)],
                oacc_hbm.at[c, pl.ds((NS - 1) * acc_a, acc_last)],
                semgA).wait()

    return edge_kernel


def kernel(entity_embed, relation_embed, edge_index, edge_type, gamma, beta,
           Wc, W1, b1, W2, b2):
    n, d = entity_embed.shape
    r = relation_embed.shape[0]
    e = edge_index.shape[1]

    assert r & (r - 1) == 0
    h = _layer_norm_tc(entity_embed, gamma, beta)

    src = edge_index.astype(jnp.int32).reshape(2 * e)
    fpk = (edge_index[1] * r + edge_type).astype(jnp.int32)

    zacc = jnp.zeros((ZROWS, d), jnp.float32)
    zcnt = jnp.zeros((CSTG,), jnp.float32)
    ones = jnp.ones((CB,), jnp.float32)

    edge_kernel = _make_edge_sc(n, d, e, r)
    acc2, cnt_flat = edge_kernel(h, src, fpk, zacc, zcnt, ones)

    cnt = cnt_flat.reshape(n, r)
    return _combine_tc(acc2, cnt, relation_embed, Wc, entity_embed)


# speedup vs baseline: 1.0450x; 1.0450x over previous
import functools

import jax
import jax.numpy as jnp
from jax import lax
from jax.experimental import pallas as pl
from jax.experimental.pallas import tpu as pltpu
from jax.experimental.pallas import tpu_sc as plsc

EPS = 1e-5

NC = 2
NS = 16

CB = 80
CSTG = 2000
ZROWS = 632


def _ln_body(x_ref, g_ref, b_ref, o_ref):
    x = x_ref[...]
    mu = jnp.mean(x, axis=-1, keepdims=True)
    xc = x - mu
    var = jnp.mean(xc * xc, axis=-1, keepdims=True)
    o_ref[...] = xc * lax.rsqrt(var + EPS) * g_ref[...] + b_ref[...]


def _layer_norm_tc(x, gamma, beta):
    n, d = x.shape
    blk = 1000
    return pl.pallas_call(
        _ln_body,
        grid=(n // blk,),
        in_specs=[
            pl.BlockSpec((blk, d), lambda i: (i, 0)),
            pl.BlockSpec((1, d), lambda i: (0, 0)),
            pl.BlockSpec((1, d), lambda i: (0, 0)),
        ],
        out_specs=pl.BlockSpec((blk, d), lambda i: (i, 0)),
        out_shape=jax.ShapeDtypeStruct((n, d), x.dtype),
    )(x, gamma.reshape(1, d), beta.reshape(1, d))


def _combine_body(acc_ref, cnt_ref, rel_ref, wc_ref, x0_ref, o_ref):
    cnt = cnt_ref[...]
    deg = jnp.sum(cnt, axis=1, keepdims=True)
    acc = acc_ref[0] + acc_ref[1]
    agg = acc + jnp.dot(cnt, rel_ref[...], preferred_element_type=jnp.float32)
    agg = agg / jnp.maximum(deg, 1.0)
    o_ref[...] = (
        jnp.dot(agg, wc_ref[...], preferred_element_type=jnp.float32)
        + x0_ref[...]
    )


def _combine_tc(acc2, cnt, rel, wc, x0):
    n, d = x0.shape
    r = rel.shape[0]
    blk = 1000
    return pl.pallas_call(
        _combine_body,
        grid=(n // blk,),
        in_specs=[
            pl.BlockSpec((NC, blk, d), lambda i: (0, i, 0)),
            pl.BlockSpec((blk, r), lambda i: (i, 0)),
            pl.BlockSpec((r, d), lambda i: (0, 0)),
            pl.BlockSpec((d, d), lambda i: (0, 0)),
            pl.BlockSpec((blk, d), lambda i: (i, 0)),
        ],
        out_specs=pl.BlockSpec((blk, d), lambda i: (i, 0)),
        out_shape=jax.ShapeDtypeStruct((n, d), jnp.float32),
    )(acc2, cnt, rel, wc, x0)


def _make_edge_sc(n, d, e, r):
    nh = n // NC
    ch = nh * r
    cnt_zpt = (ch + NS + NS * 8 - 1) // (NS * 8) * 8
    cnt_words = NS * cnt_zpt
    cnt_opt = ch // NS
    acc_a = (n // NS) // 8 * 8
    acc_last = n - acc_a * (NS - 1)
    ept = e // (NC * NS)
    nch = ept // CB
    npair = nch // 2

    mesh = plsc.VectorSubcoreMesh(core_axis_name="c", subcore_axis_name="s",
                                  num_cores=NC)

    @functools.partial(
        pl.kernel,
        out_type=(
            jax.ShapeDtypeStruct((NC, n, d), jnp.float32),
            jax.ShapeDtypeStruct((NC * ch,), jnp.float32),
        ),
        mesh=mesh,
        scratch_types=[
            pltpu.VMEM((CB,), jnp.int32),
            pltpu.VMEM((CB,), jnp.int32),
            pltpu.VMEM((CB,), jnp.int32),
            pltpu.VMEM((CB,), jnp.int32),
            pltpu.VMEM((CB,), jnp.int32),
            pltpu.VMEM((CB,), jnp.int32),
            pltpu.VMEM((CB,), jnp.int32),
            pltpu.VMEM((CB,), jnp.int32),
            pltpu.VMEM((CB,), jnp.int32),
            pltpu.VMEM((CB,), jnp.int32),
            pltpu.VMEM((CB,), jnp.int32),
            pltpu.VMEM((CB,), jnp.int32),
            pltpu.VMEM((CB,), jnp.int32),
            pltpu.VMEM((CB,), jnp.int32),
            pltpu.VMEM((CB, d), jnp.float32),
            pltpu.VMEM((CB, d), jnp.float32),
            pltpu.VMEM((CSTG,), jnp.float32),
            pltpu.VMEM((CSTG,), jnp.float32),
            pltpu.VMEM((CB,), jnp.float32),
            pltpu.VMEM_SHARED((n, d), jnp.float32),
            pltpu.VMEM_SHARED((cnt_words,), jnp.float32),
            pltpu.SemaphoreType.DMA,
            pltpu.SemaphoreType.DMA,
            pltpu.SemaphoreType.DMA,
            pltpu.SemaphoreType.DMA,
            pltpu.SemaphoreType.DMA,
        ],
    )
    def edge_kernel(h_hbm, ei_hbm, et_hbm, zacc_hbm, zcnt_hbm,
                    ones_hbm, oacc_hbm, ocnt_hbm,
                    sidxA, sidxB, didxA, didxB, eidxA, eidxB,
                    f2A, f2B, dmA, dmB, emA, emB, fmA, fmB,
                    rowsA, rowsB, cbuf, cbuf2, ones_v,
                    acc_sh, cnt_sh, semi, semgA, semgB, semsA, semsB):
        c = lax.axis_index("c")
        s = lax.axis_index("s")
        dump_cnt = ch + s
        lo = c * nh
        tbase = (c * NS + s) * ept
        mbase = ((1 - c) * NS + s) * ept

        def stage(ch_i, si, di, ei, dm, em):
            eb = tbase + ch_i * CB
            mb = mbase + ch_i * CB
            return [
                pltpu.async_copy(ei_hbm.at[pl.ds(eb, CB)], si, semi),
                pltpu.async_copy(ei_hbm.at[pl.ds(e + eb, CB)], di, semi),
                pltpu.async_copy(et_hbm.at[pl.ds(eb, CB)], ei, semi),
                pltpu.async_copy(ei_hbm.at[pl.ds(e + mb, CB)], dm, semi),
                pltpu.async_copy(et_hbm.at[pl.ds(mb, CB)], em, semi),
            ]

        def build(di, ei, f2):
            for k in range(CB // 16):
                sl = pl.ds(k * 16, 16)
                t = di[sl] - lo
                inb = (t >= 0) & (t < nh)
                f2[sl] = jnp.where(inb, t * r + ei[sl], dump_cnt)

        def fire_gather(si, rows, semg):
            return pltpu.async_copy(h_hbm.at[si], rows, semg)

        def fire_scatter(rows, di, f2, fm, sems):
            return [
                pltpu.async_copy(rows, acc_sh.at[di], sems, add=True),
                pltpu.async_copy(ones_v, cnt_sh.at[f2], sems, add=True),
                pltpu.async_copy(ones_v, cnt_sh.at[fm], sems, add=True),
            ]

        stg0 = stage(0, sidxA, didxA, eidxA, dmA, emA)

        @pl.when(s < NS - 1)
        def _():
            pltpu.async_copy(zacc_hbm, acc_sh.at[pl.ds(s * acc_a, ZROWS)],
                             semsA)

        @pl.when(s == NS - 1)
        def _():
            pltpu.async_copy(zacc_hbm, acc_sh.at[pl.ds((NS - 1) * acc_a,
                                                       ZROWS)], semsA)
            pltpu.async_copy(zacc_hbm.at[pl.ds(0, acc_last - ZROWS + 8)],
                             acc_sh.at[pl.ds((NS - 1) * acc_a + ZROWS - 8,
                                             acc_last - ZROWS + 8)], semsA)

        pltpu.sync_copy(zcnt_hbm, cbuf)
        nzc = cnt_zpt // CSTG
        zrem = cnt_zpt - nzc * CSTG
        zdesc = [
            pltpu.async_copy(
                cbuf, cnt_sh.at[pl.ds(s * cnt_zpt + p * CSTG, CSTG)], semsB)
            for p in range(nzc)
        ]
        if zrem:
            zdesc.append(pltpu.async_copy(
                cbuf.at[pl.ds(0, zrem)],
                cnt_sh.at[pl.ds(s * cnt_zpt + nzc * CSTG, zrem)], semsB))
        pltpu.sync_copy(ones_hbm, ones_v)
        for dsc in stg0:
            dsc.wait()
        fire_gather(sidxA, rowsA, semgA)
        pltpu.make_async_copy(zacc_hbm, acc_sh.at[pl.ds(s * acc_a, ZROWS)],
                              semsA).wait()

        @pl.when(s == NS - 1)
        def _():
            pltpu.make_async_copy(
                zacc_hbm.at[pl.ds(0, acc_last - ZROWS + 8)],
                acc_sh.at[pl.ds((NS - 1) * acc_a + ZROWS - 8,
                                acc_last - ZROWS + 8)], semsA).wait()

        for dsc in zdesc:
            dsc.wait()
        plsc.subcore_barrier()

        def body(i, _):
            even = 2 * i
            stg_o = stage(even + 1, sidxB, didxB, eidxB, dmB, emB)
            build(didxA, eidxA, f2A)
            build(dmA, emA, fmA)
            pltpu.make_async_copy(h_hbm.at[sidxA], rowsA, semgA).wait()
            sc_e = fire_scatter(rowsA, didxA, f2A, fmA, semsA)
            for dsc in stg_o:
                dsc.wait()
            build(didxB, eidxB, f2B)
            build(dmB, emB, fmB)
            fire_gather(sidxB, rowsB, semgB)
            for dsc in sc_e:
                dsc.wait()

            @pl.when(i < npair - 1)
            def _():
                for dsc in stage(even + 2, sidxA, didxA, eidxA, dmA, emA):
                    dsc.wait()

            pltpu.make_async_copy(h_hbm.at[sidxB], rowsB, semgB).wait()
            sc_o = fire_scatter(rowsB, didxB, f2B, fmB, semsB)

            @pl.when(i < npair - 1)
            def _():
                fire_gather(sidxA, rowsA, semgA)

            for dsc in sc_o:
                dsc.wait()
            return 0

        lax.fori_loop(0, npair, body, 0)

        if nch % 2:
            for dsc in stage(nch - 1, sidxA, didxA, eidxA, dmA, emA):
                dsc.wait()
            build(didxA, eidxA, f2A)
            build(dmA, emA, fmA)
            fire_gather(sidxA, rowsA, semgA).wait()
            for dsc in fire_scatter(rowsA, didxA, f2A, fmA, semsA):
                dsc.wait()

        plsc.subcore_barrier()

        @pl.when(s < NS - 1)
        def _():
            pltpu.async_copy(acc_sh.at[pl.ds(s * acc_a, acc_a)],
                             oacc_hbm.at[c, pl.ds(s * acc_a, acc_a)], semgA)

        @pl.when(s == NS - 1)
        def _():
            pltpu.async_copy(acc_sh.at[pl.ds((NS - 1) * acc_a, acc_last)],
                             oacc_hbm.at[c, pl.ds((NS - 1) * acc_a,
                                                  acc_last)], semgA)

        npc = cnt_opt // CSTG
        wdesc = [None, None]
        for p in range(npc):
            buf = cbuf if p % 2 == 0 else cbuf2
            if wdesc[p % 2] is not None:
                wdesc[p % 2].wait()
            pltpu.sync_copy(
                cnt_sh.at[pl.ds(s * cnt_opt + p * CSTG, CSTG)], buf)
            wdesc[p % 2] = pltpu.async_copy(
                buf,
                ocnt_hbm.at[pl.ds(c * ch + s * cnt_opt + p * CSTG, CSTG)],
                semgB)
        for dsc in wdesc:
            if dsc is not None:
                dsc.wait()

        @pl.when(s < NS - 1)
        def _():
            pltpu.make_async_copy(
                acc_sh.at[pl.ds(s * acc_a, acc_a)],
                oacc_hbm.at[c, pl.ds(s * acc_a, acc_a)], semgA).wait()

        @pl.when(s == NS - 1)
        def _():
            pltpu.make_async_copy(
                acc_sh.at[pl.ds((NS - 1) * acc_a, acc_last)],
                oacc_hbm.at[c, pl.ds((NS - 1) * acc_a, acc_last)],
                semgA).wait()

    return edge_kernel


def kernel(entity_embed, relation_embed, edge_index, edge_type, gamma, beta,
           Wc, W1, b1, W2, b2):
    n, d = entity_embed.shape
    r = relation_embed.shape[0]
    e = edge_index.shape[1]

    h = _layer_norm_tc(entity_embed, gamma, beta)

    ei = edge_index.astype(jnp.int32).reshape(2 * e)
    et = edge_type.astype(jnp.int32)

    zacc = jnp.zeros((ZROWS, d), jnp.float32)
    zcnt = jnp.zeros((CSTG,), jnp.float32)
    ones = jnp.ones((CB,), jnp.float32)

    edge_kernel = _make_edge_sc(n, d, e, r)
    acc2, cnt_flat = edge_kernel(h, ei, et, zacc, zcnt, ones)

    cnt = cnt_flat.reshape(n, r)
    return _combine_tc(acc2, cnt, relation_embed, Wc, entity_embed)
